# Initial kernel scaffold; baseline (speedup 1.0000x reference)
import jax
import jax.numpy as jnp
from jax.experimental import pallas as pl


def kernel(x, adj_t, W0, a_src0, a_dst0, b0, g0, be0, W1, a_src1, a_dst1, b1, g1, be1, W2, a_src2, a_dst2, b2, bias_last):
    raise NotImplementedError("write your pallas kernel here")


# speedup vs baseline: 10.7000x; 10.7000x over previous
import functools

import jax
import jax.numpy as jnp
from jax import lax
from jax.experimental import pallas as pl
from jax.experimental.pallas import tpu as pltpu
from jax.experimental.pallas import tpu_sc as plsc

N = 10000
E = 160000
D_IN = 256
HID = 128
HEADS = 8
OUT = 256
HC = HEADS * HID

NP = 10240
EP = 172032
EPT = EP // 16
VE = EPT // 16
SB = EPT // 128
EPH = EP // 2
EPT2 = EPH // 16
VE2 = EPT2 // 16

F32 = jnp.float32
I32 = jnp.int32


def _mesh():
    return plsc.VectorSubcoreMesh(core_axis_name="c", subcore_axis_name="s",
                                  num_cores=2, num_subcores=16)


def _make_alpha_kernel(heads):
    hps = max(heads // 2, 1)
    ept = EPT if heads == 8 else EPT2
    ve = VE if heads == 8 else VE2

    def body(src_h, dst_h, ast_h, adt_h, idm_h, ext_h, st_h,
             srcb, dstb, asr, adr, sloc, exb, zsl, idxv, ssh):
        c = lax.axis_index("c")
        sid = lax.axis_index("s")
        base = sid * ept if heads == 8 else c * EPH + sid * ept
        pltpu.sync_copy(src_h.at[pl.ds(base, ept)], srcb)
        pltpu.sync_copy(dst_h.at[pl.ds(base, ept)], dstb)
        pltpu.sync_copy(idm_h, idxv)

        def zero_z(i, _):
            for kk in range(8):
                zsl[i, pl.ds(kk * 16, 16)] = jnp.zeros((16,), F32)
            return _
        lax.fori_loop(0, 8, zero_z, 0)

        for hh in range(hps):
            head = c * hps + hh if heads == 8 else 0
            srow = head if heads == 8 else c
            pltpu.sync_copy(ast_h.at[head], asr)
            pltpu.sync_copy(adt_h.at[head], adr)

            def zero_s(i, _):
                for kk in range(8):
                    sloc[i, pl.ds(kk * 16, 16)] = jnp.zeros((16,), F32)
                return _
            lax.fori_loop(0, 80, zero_s, 0)

            def edge(i, _):
                s16 = srcb[pl.ds(i * 16, 16)]
                d16 = dstb[pl.ds(i * 16, 16)]
                si = [lax.shift_right_logical(s16, 7),
                      lax.bitwise_and(s16, 127)]
                di = [lax.shift_right_logical(d16, 7),
                      lax.bitwise_and(d16, 127)]
                av = plsc.load_gather(asr, si)
                dv = plsc.load_gather(adr, di)
                e = av + dv
                e = jnp.where(e < 0, e * F32(0.2), e)
                ex = jnp.exp(e)
                exb[pl.ds(i * 16, 16)] = ex
                plsc.addupdate_scatter(sloc, di, ex)
                return _
            lax.fori_loop(0, ve, edge, 0)

            pltpu.sync_copy(exb, ext_h.at[pl.ds(head * EP + base, ept)])

            @pl.when(sid < 10)
            def _():
                pltpu.sync_copy(zsl, ssh.at[pl.ds(sid * 8, 8)])
            plsc.subcore_barrier()
            pltpu.sync_copy(sloc, ssh.at[idxv], add=True)
            plsc.subcore_barrier()

            @pl.when(sid < 10)
            def _():
                pltpu.sync_copy(ssh.at[pl.ds(sid * 8, 8)],
                                st_h.at[srow, pl.ds(sid * 8, 8)])
            plsc.subcore_barrier()

    srows = heads if heads == 8 else 2
    return pl.kernel(
        body,
        out_type=[jax.ShapeDtypeStruct((heads * EP,), F32),
                  jax.ShapeDtypeStruct((srows, 80, 128), F32)],
        mesh=_mesh(),
        compiler_params=pltpu.CompilerParams(needs_layout_passes=False),
        scratch_types=[
            pltpu.VMEM((ept,), I32),
            pltpu.VMEM((ept,), I32),
            pltpu.VMEM((80, 128), F32),
            pltpu.VMEM((80, 128), F32),
            pltpu.VMEM((80, 128), F32),
            pltpu.VMEM((ept,), F32),
            pltpu.VMEM((8, 128), F32),
            pltpu.VMEM((80,), I32),
            pltpu.VMEM_SHARED((80, 128), F32),
        ],
    )


def _make_agg_kernel(heads, ch):
    hps = max(heads // 2, 1)
    accr = NP if heads == 8 else NP // 2
    rpt = accr // 16
    nz = rpt // 64
    cw = ch // 128

    def body(src_h, dst_h, ext_h, rt_h, htab_h, agg_h,
             s128, d128, e128, al128, ridb, dll, rrow, rows, sem, agg_acc):
        c = lax.axis_index("c")
        sid = lax.axis_index("s")
        ebase = sid * EPT
        nbase = c * accr if heads == 1 else 0

        for hh in range(hps):
            head = c * hps + hh if heads == 8 else 0

            def zero_rows(i, _):
                for g in range(cw):
                    for kk in range(8):
                        rows[i, g, pl.ds(kk * 16, 16)] = jnp.zeros((16,), F32)
                return _
            lax.fori_loop(0, 64, zero_rows, 0)
            for k in range(nz):
                pltpu.sync_copy(rows.at[pl.ds(0, 64)],
                                agg_acc.at[pl.ds(sid * rpt + k * 64, 64)])
            pltpu.sync_copy(rt_h.at[head], rrow)
            plsc.subcore_barrier()

            def batch(j, _):
                eoff = ebase + j * 128
                pltpu.sync_copy(src_h.at[pl.ds(eoff, 128)], s128)
                pltpu.sync_copy(dst_h.at[pl.ds(eoff, 128)], d128)
                pltpu.sync_copy(ext_h.at[pl.ds(head * EP + eoff, 128)], e128)
                for kk in range(8):
                    s16 = s128[pl.ds(kk * 16, 16)]
                    d16 = d128[pl.ds(kk * 16, 16)]
                    ex16 = e128[pl.ds(kk * 16, 16)]
                    r16 = plsc.load_gather(
                        rrow, [lax.shift_right_logical(d16, 7),
                               lax.bitwise_and(d16, 127)])
                    a16 = ex16 * r16
                    rid = s16 * heads + head
                    if heads == 1:
                        inb = (d16 >= nbase) & (d16 < nbase + accr)
                        a16 = jnp.where(inb, a16, F32(0.0))
                        dloc = jnp.clip(d16 - nbase, 0, accr - 1)
                    else:
                        dloc = d16
                    al128[pl.ds(kk * 16, 16)] = a16
                    dll[pl.ds(kk * 16, 16)] = dloc
                    ridb[pl.ds(kk * 16, 16)] = rid
                pltpu.async_copy(htab_h.at[ridb], rows, sem).wait()

                def scale(i, _2):
                    av = al128[pl.ds(i * 16, 16)]
                    for ll in range(16):
                        a = av[ll]
                        r = i * 16 + ll
                        for g in range(cw):
                            for kc in range(8):
                                rows[r, g, pl.ds(kc * 16, 16)] = (
                                    rows[r, g, pl.ds(kc * 16, 16)] * a)
                    return _2
                lax.fori_loop(0, 8, scale, 0)
                pltpu.sync_copy(rows, agg_acc.at[dll], add=True)
                return _
            lax.fori_loop(0, SB, batch, 0)
            plsc.subcore_barrier()
            if heads == 8:
                pltpu.sync_copy(agg_acc.at[pl.ds(sid * rpt, rpt), 0],
                                agg_h.at[head, pl.ds(sid * rpt, rpt)])
            else:
                pltpu.sync_copy(agg_acc.at[pl.ds(sid * rpt, rpt)],
                                agg_h.at[pl.ds(nbase + sid * rpt, rpt)])
            plsc.subcore_barrier()

    out_shape = (jax.ShapeDtypeStruct((heads, NP, ch), F32) if heads == 8
                 else jax.ShapeDtypeStruct((NP, cw, 128), F32))
    return pl.kernel(
        body,
        out_type=[out_shape],
        mesh=_mesh(),
        compiler_params=pltpu.CompilerParams(needs_layout_passes=False),
        scratch_types=[
            pltpu.VMEM((128,), I32),
            pltpu.VMEM((128,), I32),
            pltpu.VMEM((128,), F32),
            pltpu.VMEM((128,), F32),
            pltpu.VMEM((128,), I32),
            pltpu.VMEM((128,), I32),
            pltpu.VMEM((80, 128), F32),
            pltpu.VMEM((128, cw, 128), F32),
            pltpu.SemaphoreType.DMA,
            pltpu.VMEM_SHARED((accr, cw, 128), F32),
        ],
    )


def _make_recip_kernel(srows, combine):
    rrows = 1 if combine else srows

    def body(s_ref, r_ref):
        s = s_ref[...]
        if combine:
            tot = (s[0] + s[1])[None]
        else:
            tot = s
        r_ref[...] = 1.0 / (tot + 1e-16)

    return pl.pallas_call(
        body,
        grid=(1,),
        in_specs=[pl.BlockSpec((srows, 80, 128), lambda i: (0, 0, 0))],
        out_specs=pl.BlockSpec((rrows, 80, 128), lambda i: (0, 0, 0)),
        out_shape=jax.ShapeDtypeStruct((rrows, 80, 128), F32),
    )


def _make_tc_layer(din, dout, heads, pre):
    BN = 256
    grid = (NP // BN,)

    def body(*refs):
        if pre:
            x_ref, ge_ref, be_ref, w_ref, as_ref, ad_ref, h_ref, at_s, at_d = refs
            z = x_ref[...]
            z = jnp.maximum(z * ge_ref[...] + be_ref[...], 0.0)
        else:
            x_ref, w_ref, as_ref, ad_ref, h_ref, at_s, at_d = refs
            z = x_ref[...]
        h = jnp.dot(z, w_ref[...], preferred_element_type=F32)
        h_ref[...] = h
        dn = (((0,), (1,)), ((), ()))
        at_s[...] = lax.dot_general(as_ref[...], h, dn,
                                    preferred_element_type=F32)
        at_d[...] = lax.dot_general(ad_ref[...], h, dn,
                                    preferred_element_type=F32)

    in_specs = [pl.BlockSpec((BN, din), lambda i: (i, 0))]
    if pre:
        in_specs += [pl.BlockSpec((1, din), lambda i: (0, 0)),
                     pl.BlockSpec((1, din), lambda i: (0, 0))]
    in_specs += [pl.BlockSpec((din, dout), lambda i: (0, 0)),
                 pl.BlockSpec((dout, heads), lambda i: (0, 0)),
                 pl.BlockSpec((dout, heads), lambda i: (0, 0))]
    return pl.pallas_call(
        body,
        grid=grid,
        in_specs=in_specs,
        out_specs=[pl.BlockSpec((BN, dout), lambda i: (i, 0)),
                   pl.BlockSpec((heads, BN), lambda i: (0, i)),
                   pl.BlockSpec((heads, BN), lambda i: (0, i))],
        out_shape=[jax.ShapeDtypeStruct((NP, dout), F32),
                   jax.ShapeDtypeStruct((heads, NP), F32),
                   jax.ShapeDtypeStruct((heads, NP), F32)],
    )


def _logsoftmax_kernel():
    BN = 256

    def body(y_ref, cb_ref, o_ref):
        y = y_ref[...] + cb_ref[...]
        m = jnp.max(y, axis=-1, keepdims=True)
        z = y - m
        s = jnp.sum(jnp.exp(z), axis=-1, keepdims=True)
        o_ref[...] = z - jnp.log(s)

    return pl.pallas_call(
        body,
        grid=(NP // BN,),
        in_specs=[pl.BlockSpec((BN, OUT), lambda i: (i, 0)),
                  pl.BlockSpec((1, OUT), lambda i: (0, 0))],
        out_specs=pl.BlockSpec((BN, OUT), lambda i: (i, 0)),
        out_shape=jax.ShapeDtypeStruct((NP, OUT), F32),
    )


def _block_diag_att(a):
    h, c = a.shape
    return (a[:, :, None] * jnp.eye(h, dtype=a.dtype)[:, None, :]).reshape(
        h * c, h)


def kernel(x, adj_t, W0, a_src0, a_dst0, b0, g0, be0,
           W1, a_src1, a_dst1, b1, g1, be1,
           W2, a_src2, a_dst2, b2, bias_last):
    adj = adj_t.astype(I32)
    loops = jnp.arange(N, dtype=I32)
    padv = jnp.full((EP - E - N,), N, dtype=I32)
    src = jnp.concatenate([adj[0], loops, padv])
    dst = jnp.concatenate([adj[1], loops, padv])
    xp = jnp.zeros((NP, D_IN), F32).at[:N].set(x)
    idm = jnp.arange(80, dtype=I32)

    bnscale = 1.0 / jnp.sqrt(jnp.float32(1.0 + 1e-5))
    ge0 = (g0 * bnscale).reshape(1, HC)
    bf0 = (b0 * g0 * bnscale + be0).reshape(1, HC)
    ge1 = (g1 * bnscale).reshape(1, HC)
    bf1 = (b1 * g1 * bnscale + be1).reshape(1, HC)
    cb = (b2 + bias_last).reshape(1, OUT)

    A_s0, A_d0 = _block_diag_att(a_src0), _block_diag_att(a_dst0)
    A_s1, A_d1 = _block_diag_att(a_src1), _block_diag_att(a_dst1)
    A_s2, A_d2 = a_src2.reshape(OUT, 1), a_dst2.reshape(OUT, 1)

    tc0 = _make_tc_layer(D_IN, HC, HEADS, pre=False)
    tc1 = _make_tc_layer(HC, HC, HEADS, pre=True)
    tc2 = _make_tc_layer(HC, OUT, 1, pre=True)
    al8 = _make_alpha_kernel(8)
    al1 = _make_alpha_kernel(1)
    ag8 = _make_agg_kernel(8, HID)
    ag1 = _make_agg_kernel(1, OUT)
    rc8 = _make_recip_kernel(8, combine=False)
    rc1 = _make_recip_kernel(2, combine=True)
    lsm = _logsoftmax_kernel()

    h0, ast0, adt0 = tc0(xp, W0, A_s0, A_d0)
    ext0, st0 = al8(src, dst, ast0.reshape(8, 80, 128),
                    adt0.reshape(8, 80, 128), idm)
    rt0 = rc8(st0)
    (agg0,) = ag8(src, dst, ext0, rt0,
                  h0.reshape(NP * HEADS, 1, HID))
    z0 = jnp.transpose(agg0, (1, 0, 2)).reshape(NP, HC)
    h1, ast1, adt1 = tc1(z0, ge0, bf0, W1, A_s1, A_d1)
    ext1, st1 = al8(src, dst, ast1.reshape(8, 80, 128),
                    adt1.reshape(8, 80, 128), idm)
    rt1 = rc8(st1)
    (agg1,) = ag8(src, dst, ext1, rt1,
                  h1.reshape(NP * HEADS, 1, HID))
    z1 = jnp.transpose(agg1, (1, 0, 2)).reshape(NP, HC)
    h2, ast2, adt2 = tc2(z1, ge1, bf1, W2, A_s2, A_d2)
    ext2, st2 = al1(src, dst, ast2.reshape(1, 80, 128),
                    adt2.reshape(1, 80, 128), idm)
    rt2 = rc1(st2)
    (agg2,) = ag1(src, dst, ext2, rt2, h2.reshape(NP, 2, 128))
    out = lsm(agg2.reshape(NP, OUT), cb)
    return out[:N]


# speedup vs baseline: 15.2411x; 1.4244x over previous
import functools

import jax
import jax.numpy as jnp
from jax import lax
from jax.experimental import pallas as pl
from jax.experimental.pallas import tpu as pltpu
from jax.experimental.pallas import tpu_sc as plsc

N = 10000
E = 160000
D_IN = 256
HID = 128
HEADS = 8
OUT = 256
HC = HEADS * HID

NP = 10240
EP = 172032
EPT = EP // 16
VE = EPT // 16
SB = EPT // 128
EPH = EP // 2
EPT2 = EPH // 16
VE2 = EPT2 // 16

F32 = jnp.float32
I32 = jnp.int32


def _mesh():
    return plsc.VectorSubcoreMesh(core_axis_name="c", subcore_axis_name="s",
                                  num_cores=2, num_subcores=16)


def _make_alpha_kernel(heads):
    hps = max(heads // 2, 1)
    ept = EPT if heads == 8 else EPT2
    ve = VE if heads == 8 else VE2

    def body(src_h, dst_h, ast_h, adt_h, idm_h, ext_h, st_h,
             srcb, dstb, asr, adr, sloc, exb, zsl, idxv, ssh):
        c = lax.axis_index("c")
        sid = lax.axis_index("s")
        base = sid * ept if heads == 8 else c * EPH + sid * ept
        pltpu.sync_copy(src_h.at[pl.ds(base, ept)], srcb)
        pltpu.sync_copy(dst_h.at[pl.ds(base, ept)], dstb)
        pltpu.sync_copy(idm_h, idxv)

        def zero_z(i, _):
            for kk in range(8):
                zsl[i, pl.ds(kk * 16, 16)] = jnp.zeros((16,), F32)
            return _
        lax.fori_loop(0, 8, zero_z, 0)

        for hh in range(hps):
            head = c * hps + hh if heads == 8 else 0
            srow = head if heads == 8 else c
            pltpu.sync_copy(ast_h.at[head], asr)
            pltpu.sync_copy(adt_h.at[head], adr)

            def zero_s(i, _):
                for kk in range(8):
                    sloc[i, pl.ds(kk * 16, 16)] = jnp.zeros((16,), F32)
                return _
            lax.fori_loop(0, 80, zero_s, 0)

            def edge(i, _):
                s16 = srcb[pl.ds(i * 16, 16)]
                d16 = dstb[pl.ds(i * 16, 16)]
                si = [lax.shift_right_logical(s16, 7),
                      lax.bitwise_and(s16, 127)]
                di = [lax.shift_right_logical(d16, 7),
                      lax.bitwise_and(d16, 127)]
                av = plsc.load_gather(asr, si)
                dv = plsc.load_gather(adr, di)
                e = av + dv
                e = jnp.where(e < 0, e * F32(0.2), e)
                ex = jnp.exp(e)
                exb[pl.ds(i * 16, 16)] = ex
                plsc.addupdate_scatter(sloc, di, ex)
                return _
            lax.fori_loop(0, ve, edge, 0)

            pltpu.sync_copy(exb, ext_h.at[pl.ds(head * EP + base, ept)])

            @pl.when(sid < 10)
            def _():
                pltpu.sync_copy(zsl, ssh.at[pl.ds(sid * 8, 8)])
            plsc.subcore_barrier()
            pltpu.sync_copy(sloc, ssh.at[idxv], add=True)
            plsc.subcore_barrier()

            @pl.when(sid < 10)
            def _():
                pltpu.sync_copy(ssh.at[pl.ds(sid * 8, 8)],
                                st_h.at[srow, pl.ds(sid * 8, 8)])
            plsc.subcore_barrier()

    srows = heads if heads == 8 else 2
    return pl.kernel(
        body,
        out_type=[jax.ShapeDtypeStruct((heads * EP,), F32),
                  jax.ShapeDtypeStruct((srows, 80, 128), F32)],
        mesh=_mesh(),
        compiler_params=pltpu.CompilerParams(needs_layout_passes=False),
        scratch_types=[
            pltpu.VMEM((ept,), I32),
            pltpu.VMEM((ept,), I32),
            pltpu.VMEM((80, 128), F32),
            pltpu.VMEM((80, 128), F32),
            pltpu.VMEM((80, 128), F32),
            pltpu.VMEM((ept,), F32),
            pltpu.VMEM((8, 128), F32),
            pltpu.VMEM((80,), I32),
            pltpu.VMEM_SHARED((80, 128), F32),
        ],
    )


def _make_agg_kernel(heads, ch):
    hps = max(heads // 2, 1)
    accr = NP if heads == 8 else NP // 2
    rpt = accr // 16
    cw = ch // 128
    br = 128 // cw
    nz = rpt // br
    pg = 512
    nb = pg // br
    npg = EPT // pg

    def body(src_h, dst_h, ext_h, rt_h, htab_h, agg_h,
             srcp, dstp, exp_, alp, ridp, dllp, rrow, rows2,
             stsem, gsem0, gsem1, wsem0, wsem1, agg_acc):
        c = lax.axis_index("c")
        sid = lax.axis_index("s")
        ebase = sid * EPT
        nbase = c * accr if heads == 1 else 0
        gsems = (gsem0, gsem1)
        wsems = (wsem0, wsem1)

        for hh in range(hps):
            head = c * hps + hh if heads == 8 else 0

            def zero_rows(i, _):
                for g in range(cw):
                    for kk in range(8):
                        rows2[0, i, g, pl.ds(kk * 16, 16)] = (
                            jnp.zeros((16,), F32))
                return _
            lax.fori_loop(0, br, zero_rows, 0)
            for k in range(nz):
                pltpu.sync_copy(rows2.at[0],
                                agg_acc.at[pl.ds(sid * rpt + k * br, br)])
            pltpu.sync_copy(rt_h.at[head], rrow)
            plsc.subcore_barrier()

            def page(jp, _):
                eoff = ebase + jp * pg
                c1 = pltpu.async_copy(src_h.at[pl.ds(eoff, pg)], srcp, stsem)
                c2 = pltpu.async_copy(dst_h.at[pl.ds(eoff, pg)], dstp, stsem)
                c3 = pltpu.async_copy(
                    ext_h.at[pl.ds(head * EP + eoff, pg)], exp_, stsem)
                c1.wait(); c2.wait(); c3.wait()

                def mkal(i, _2):
                    s16 = srcp[pl.ds(i * 16, 16)]
                    d16 = dstp[pl.ds(i * 16, 16)]
                    ex16 = exp_[pl.ds(i * 16, 16)]
                    r16 = plsc.load_gather(
                        rrow, [lax.shift_right_logical(d16, 7),
                               lax.bitwise_and(d16, 127)])
                    a16 = ex16 * r16
                    rid = s16 * heads + head
                    if heads == 1:
                        inb = (d16 >= nbase) & (d16 < nbase + accr)
                        a16 = jnp.where(inb, a16, F32(0.0))
                        dloc = jnp.clip(d16 - nbase, 0, accr - 1)
                    else:
                        dloc = d16
                    alp[pl.ds(i * 16, 16)] = a16
                    kb = i // (br // 16)
                    ko = (i % (br // 16)) * 16
                    ridp[kb, pl.ds(ko, 16)] = rid
                    dllp[kb, pl.ds(ko, 16)] = dloc
                    return _2
                lax.fori_loop(0, pg // 16, mkal, 0)

                gds = [None] * nb
                wds = [None] * nb
                gds[0] = pltpu.async_copy(
                    htab_h.at[ridp.at[0]], rows2.at[0], gsems[0])
                for k in range(nb):
                    b = k % 2
                    if k + 1 < nb:
                        if wds[k - 1] is not None:
                            wds[k - 1].wait()
                        gds[k + 1] = pltpu.async_copy(
                            htab_h.at[ridp.at[k + 1]], rows2.at[1 - b],
                            gsems[1 - b])
                    gds[k].wait()

                    def scale(i, _2, k=k, b=b):
                        av = alp[pl.ds(k * br + i * 16, 16)]
                        for ll in range(16):
                            a = av[ll]
                            r = i * 16 + ll
                            for g in range(cw):
                                for kc in range(8):
                                    rows2[b, r, g, pl.ds(kc * 16, 16)] = (
                                        rows2[b, r, g, pl.ds(kc * 16, 16)]
                                        * a)
                        return _2
                    lax.fori_loop(0, br // 16, scale, 0)
                    wds[k] = pltpu.async_copy(
                        rows2.at[b], agg_acc.at[dllp.at[k]], wsems[b],
                        add=True)
                wds[nb - 2].wait()
                wds[nb - 1].wait()
                return _
            lax.fori_loop(0, npg, page, 0)
            plsc.subcore_barrier()
            if heads == 8:
                pltpu.sync_copy(agg_acc.at[pl.ds(sid * rpt, rpt), 0],
                                agg_h.at[head, pl.ds(sid * rpt, rpt)])
            else:
                pltpu.sync_copy(agg_acc.at[pl.ds(sid * rpt, rpt)],
                                agg_h.at[pl.ds(nbase + sid * rpt, rpt)])
            plsc.subcore_barrier()

    out_shape = (jax.ShapeDtypeStruct((heads, NP, ch), F32) if heads == 8
                 else jax.ShapeDtypeStruct((NP, cw, 128), F32))
    return pl.kernel(
        body,
        out_type=[out_shape],
        mesh=_mesh(),
        compiler_params=pltpu.CompilerParams(needs_layout_passes=False),
        scratch_types=[
            pltpu.VMEM((pg,), I32),
            pltpu.VMEM((pg,), I32),
            pltpu.VMEM((pg,), F32),
            pltpu.VMEM((pg,), F32),
            pltpu.VMEM((nb, br), I32),
            pltpu.VMEM((nb, br), I32),
            pltpu.VMEM((80, 128), F32),
            pltpu.VMEM((2, br, cw, 128), F32),
            pltpu.SemaphoreType.DMA,
            pltpu.SemaphoreType.DMA,
            pltpu.SemaphoreType.DMA,
            pltpu.SemaphoreType.DMA,
            pltpu.SemaphoreType.DMA,
            pltpu.VMEM_SHARED((accr, cw, 128), F32),
        ],
    )


def _make_recip_kernel(srows, combine):
    rrows = 1 if combine else srows

    def body(s_ref, r_ref):
        s = s_ref[...]
        if combine:
            tot = (s[0] + s[1])[None]
        else:
            tot = s
        r_ref[...] = 1.0 / (tot + 1e-16)

    return pl.pallas_call(
        body,
        grid=(1,),
        in_specs=[pl.BlockSpec((srows, 80, 128), lambda i: (0, 0, 0))],
        out_specs=pl.BlockSpec((rrows, 80, 128), lambda i: (0, 0, 0)),
        out_shape=jax.ShapeDtypeStruct((rrows, 80, 128), F32),
    )


def _make_tc_layer(din, dout, heads, pre):
    BN = 256
    grid = (NP // BN,)

    def body(*refs):
        if pre:
            x_ref, ge_ref, be_ref, w_ref, as_ref, ad_ref, h_ref, at_s, at_d = refs
            z = x_ref[...]
            z = jnp.maximum(z * ge_ref[...] + be_ref[...], 0.0)
        else:
            x_ref, w_ref, as_ref, ad_ref, h_ref, at_s, at_d = refs
            z = x_ref[...]
        h = jnp.dot(z, w_ref[...], preferred_element_type=F32)
        h_ref[...] = h
        dn = (((0,), (1,)), ((), ()))
        at_s[...] = lax.dot_general(as_ref[...], h, dn,
                                    preferred_element_type=F32)
        at_d[...] = lax.dot_general(ad_ref[...], h, dn,
                                    preferred_element_type=F32)

    in_specs = [pl.BlockSpec((BN, din), lambda i: (i, 0))]
    if pre:
        in_specs += [pl.BlockSpec((1, din), lambda i: (0, 0)),
                     pl.BlockSpec((1, din), lambda i: (0, 0))]
    in_specs += [pl.BlockSpec((din, dout), lambda i: (0, 0)),
                 pl.BlockSpec((dout, heads), lambda i: (0, 0)),
                 pl.BlockSpec((dout, heads), lambda i: (0, 0))]
    return pl.pallas_call(
        body,
        grid=grid,
        in_specs=in_specs,
        out_specs=[pl.BlockSpec((BN, dout), lambda i: (i, 0)),
                   pl.BlockSpec((heads, BN), lambda i: (0, i)),
                   pl.BlockSpec((heads, BN), lambda i: (0, i))],
        out_shape=[jax.ShapeDtypeStruct((NP, dout), F32),
                   jax.ShapeDtypeStruct((heads, NP), F32),
                   jax.ShapeDtypeStruct((heads, NP), F32)],
    )


def _logsoftmax_kernel():
    BN = 256

    def body(y_ref, cb_ref, o_ref):
        y = y_ref[...] + cb_ref[...]
        m = jnp.max(y, axis=-1, keepdims=True)
        z = y - m
        s = jnp.sum(jnp.exp(z), axis=-1, keepdims=True)
        o_ref[...] = z - jnp.log(s)

    return pl.pallas_call(
        body,
        grid=(NP // BN,),
        in_specs=[pl.BlockSpec((BN, OUT), lambda i: (i, 0)),
                  pl.BlockSpec((1, OUT), lambda i: (0, 0))],
        out_specs=pl.BlockSpec((BN, OUT), lambda i: (i, 0)),
        out_shape=jax.ShapeDtypeStruct((NP, OUT), F32),
    )


def _block_diag_att(a):
    h, c = a.shape
    return (a[:, :, None] * jnp.eye(h, dtype=a.dtype)[:, None, :]).reshape(
        h * c, h)


def kernel(x, adj_t, W0, a_src0, a_dst0, b0, g0, be0,
           W1, a_src1, a_dst1, b1, g1, be1,
           W2, a_src2, a_dst2, b2, bias_last):
    adj = adj_t.astype(I32)
    loops = jnp.arange(N, dtype=I32)
    padv = jnp.full((EP - E - N,), N, dtype=I32)
    src = jnp.concatenate([adj[0], loops, padv])
    dst = jnp.concatenate([adj[1], loops, padv])
    xp = jnp.zeros((NP, D_IN), F32).at[:N].set(x)
    idm = jnp.arange(80, dtype=I32)

    bnscale = 1.0 / jnp.sqrt(jnp.float32(1.0 + 1e-5))
    ge0 = (g0 * bnscale).reshape(1, HC)
    bf0 = (b0 * g0 * bnscale + be0).reshape(1, HC)
    ge1 = (g1 * bnscale).reshape(1, HC)
    bf1 = (b1 * g1 * bnscale + be1).reshape(1, HC)
    cb = (b2 + bias_last).reshape(1, OUT)

    A_s0, A_d0 = _block_diag_att(a_src0), _block_diag_att(a_dst0)
    A_s1, A_d1 = _block_diag_att(a_src1), _block_diag_att(a_dst1)
    A_s2, A_d2 = a_src2.reshape(OUT, 1), a_dst2.reshape(OUT, 1)

    tc0 = _make_tc_layer(D_IN, HC, HEADS, pre=False)
    tc1 = _make_tc_layer(HC, HC, HEADS, pre=True)
    tc2 = _make_tc_layer(HC, OUT, 1, pre=True)
    al8 = _make_alpha_kernel(8)
    al1 = _make_alpha_kernel(1)
    ag8 = _make_agg_kernel(8, HID)
    ag1 = _make_agg_kernel(1, OUT)
    rc8 = _make_recip_kernel(8, combine=False)
    rc1 = _make_recip_kernel(2, combine=True)
    lsm = _logsoftmax_kernel()

    h0, ast0, adt0 = tc0(xp, W0, A_s0, A_d0)
    ext0, st0 = al8(src, dst, ast0.reshape(8, 80, 128),
                    adt0.reshape(8, 80, 128), idm)
    rt0 = rc8(st0)
    (agg0,) = ag8(src, dst, ext0, rt0,
                  h0.reshape(NP * HEADS, 1, HID))
    z0 = jnp.transpose(agg0, (1, 0, 2)).reshape(NP, HC)
    h1, ast1, adt1 = tc1(z0, ge0, bf0, W1, A_s1, A_d1)
    ext1, st1 = al8(src, dst, ast1.reshape(8, 80, 128),
                    adt1.reshape(8, 80, 128), idm)
    rt1 = rc8(st1)
    (agg1,) = ag8(src, dst, ext1, rt1,
                  h1.reshape(NP * HEADS, 1, HID))
    z1 = jnp.transpose(agg1, (1, 0, 2)).reshape(NP, HC)
    h2, ast2, adt2 = tc2(z1, ge1, bf1, W2, A_s2, A_d2)
    ext2, st2 = al1(src, dst, ast2.reshape(1, 80, 128),
                    adt2.reshape(1, 80, 128), idm)
    rt2 = rc1(st2)
    (agg2,) = ag1(src, dst, ext2, rt2, h2.reshape(NP, 2, 128))
    out = lsm(agg2.reshape(NP, OUT), cb)
    return out[:N]


# speedup vs baseline: 15.7930x; 1.0362x over previous
import functools

import jax
import jax.numpy as jnp
from jax import lax
from jax.experimental import pallas as pl
from jax.experimental.pallas import tpu as pltpu
from jax.experimental.pallas import tpu_sc as plsc

N = 10000
E = 160000
D_IN = 256
HID = 128
HEADS = 8
OUT = 256
HC = HEADS * HID

NP = 10240
EP = 172032
EPT = EP // 16
VE = EPT // 16
SB = EPT // 128
EPH = EP // 2
EPT2 = EPH // 16
VE2 = EPT2 // 16

F32 = jnp.float32
I32 = jnp.int32


def _mesh():
    return plsc.VectorSubcoreMesh(core_axis_name="c", subcore_axis_name="s",
                                  num_cores=2, num_subcores=16)


def _make_alpha_kernel(heads):
    hps = max(heads // 2, 1)
    ept = EPT if heads == 8 else EPT2
    ve = VE if heads == 8 else VE2

    def body(src_h, dst_h, ast_h, adt_h, idm_h, ext_h, st_h,
             srcb, dstb, asr, adr, sloc, exb, zsl, idxv, ssh):
        c = lax.axis_index("c")
        sid = lax.axis_index("s")
        base = sid * ept if heads == 8 else c * EPH + sid * ept
        pltpu.sync_copy(src_h.at[pl.ds(base, ept)], srcb)
        pltpu.sync_copy(dst_h.at[pl.ds(base, ept)], dstb)
        pltpu.sync_copy(idm_h, idxv)

        def zero_z(i, _):
            for kk in range(8):
                zsl[i, pl.ds(kk * 16, 16)] = jnp.zeros((16,), F32)
            return _
        lax.fori_loop(0, 8, zero_z, 0)

        def head_iter(hh, _h):
            head = c * hps + hh if heads == 8 else 0
            srow = head if heads == 8 else c
            pltpu.sync_copy(ast_h.at[head], asr)
            pltpu.sync_copy(adt_h.at[head], adr)

            def zero_s(i, _):
                for kk in range(8):
                    sloc[i, pl.ds(kk * 16, 16)] = jnp.zeros((16,), F32)
                return _
            lax.fori_loop(0, 80, zero_s, 0)

            def edge(i, _):
                s16 = srcb[pl.ds(i * 16, 16)]
                d16 = dstb[pl.ds(i * 16, 16)]
                si = [lax.shift_right_logical(s16, 7),
                      lax.bitwise_and(s16, 127)]
                di = [lax.shift_right_logical(d16, 7),
                      lax.bitwise_and(d16, 127)]
                av = plsc.load_gather(asr, si)
                dv = plsc.load_gather(adr, di)
                e = av + dv
                e = jnp.where(e < 0, e * F32(0.2), e)
                ex = jnp.exp(e)
                exb[pl.ds(i * 16, 16)] = ex
                plsc.addupdate_scatter(sloc, di, ex)
                return _
            lax.fori_loop(0, ve, edge, 0)

            pltpu.sync_copy(exb, ext_h.at[pl.ds(head * EP + base, ept)])

            @pl.when(sid < 10)
            def _():
                pltpu.sync_copy(zsl, ssh.at[pl.ds(sid * 8, 8)])
            plsc.subcore_barrier()
            pltpu.sync_copy(sloc, ssh.at[idxv], add=True)
            plsc.subcore_barrier()

            @pl.when(sid < 10)
            def _():
                pltpu.sync_copy(ssh.at[pl.ds(sid * 8, 8)],
                                st_h.at[srow, pl.ds(sid * 8, 8)])
            plsc.subcore_barrier()

    srows = heads if heads == 8 else 2
    return pl.kernel(
        body,
        out_type=[jax.ShapeDtypeStruct((heads * EP,), F32),
                  jax.ShapeDtypeStruct((srows, 80, 128), F32)],
        mesh=_mesh(),
        compiler_params=pltpu.CompilerParams(needs_layout_passes=False),
        scratch_types=[
            pltpu.VMEM((ept,), I32),
            pltpu.VMEM((ept,), I32),
            pltpu.VMEM((80, 128), F32),
            pltpu.VMEM((80, 128), F32),
            pltpu.VMEM((80, 128), F32),
            pltpu.VMEM((ept,), F32),
            pltpu.VMEM((8, 128), F32),
            pltpu.VMEM((80,), I32),
            pltpu.VMEM_SHARED((80, 128), F32),
        ],
    )


def _make_agg_kernel(heads, ch):
    hps = max(heads // 2, 1)
    accr = NP if heads == 8 else NP // 2
    rpt = accr // 16
    cw = ch // 128
    br = 128 // cw
    nz = rpt // br
    pg = 512
    nb = pg // br
    npg = EPT // pg
    npair = (npg - 1) // 2

    def body(src_h, dst_h, ext_h, htab_h, agg_h,
             srcp, dstp, exp_, alp, ridp, dllp, rows2,
             stsem, gsem0, gsem1, wsem0, wsem1, agg_acc):
        c = lax.axis_index("c")
        sid = lax.axis_index("s")
        ebase = sid * EPT
        nbase = c * accr if heads == 1 else 0
        gsems = (gsem0, gsem1)
        wsems = (wsem0, wsem1)

        def issue_stage(pp, slot):
            eoff = ebase + pp * pg
            so = slot * pg
            pltpu.async_copy(src_h.at[pl.ds(eoff, pg)],
                             srcp.at[pl.ds(so, pg)], stsem)
            pltpu.async_copy(dst_h.at[pl.ds(eoff, pg)],
                             dstp.at[pl.ds(so, pg)], stsem)

        def issue_stage_ex(pp, slot, head):
            eoff = ebase + pp * pg
            so = slot * pg
            pltpu.async_copy(ext_h.at[pl.ds(head * EP + eoff, pg)],
                             exp_.at[pl.ds(so, pg)], stsem)

        def wait_stage(slot):
            so = slot * pg
            pltpu.make_async_copy(src_h.at[pl.ds(0, pg)],
                                  srcp.at[pl.ds(so, pg)], stsem).wait()
            pltpu.make_async_copy(dst_h.at[pl.ds(0, pg)],
                                  dstp.at[pl.ds(so, pg)], stsem).wait()
            pltpu.make_async_copy(ext_h.at[pl.ds(0, pg)],
                                  exp_.at[pl.ds(so, pg)], stsem).wait()

        def wait_w(rb):
            pltpu.make_async_copy(rows2.at[rb], agg_acc.at[ridp.at[0, 0]],
                                  wsems[rb]).wait()

        def mkal(slot, head):
            so = slot * pg

            def step(i, _):
                s16 = srcp[pl.ds(so + i * 16, 16)]
                d16 = dstp[pl.ds(so + i * 16, 16)]
                a16 = exp_[pl.ds(so + i * 16, 16)]
                rid = s16 * heads + head
                if heads == 1:
                    inb = (d16 >= nbase) & (d16 < nbase + accr)
                    a16 = jnp.where(inb, a16, F32(0.0))
                    dloc = jnp.clip(d16 - nbase, 0, accr - 1)
                else:
                    dloc = d16
                alp[slot * 4 + i // 8, pl.ds((i % 8) * 16, 16)] = a16
                kb = i // (br // 16)
                ko = (i % (br // 16)) * 16
                ridp[slot, kb, pl.ds(ko, 16)] = rid
                dllp[slot, kb, pl.ds(ko, 16)] = dloc
                return _
            lax.fori_loop(0, pg // 16, step, 0)

        def run_page(slot, head, first):
            wds = [None] * nb
            if not first:
                wait_w(0)
            gds = [None] * nb
            gds[0] = pltpu.async_copy(
                htab_h.at[ridp.at[slot, 0]], rows2.at[0], gsems[0])
            for k in range(nb):
                rb = k % 2
                if k + 1 < nb:
                    nrb = 1 - rb
                    if k + 1 >= 2:
                        wds[k - 1].wait()
                    elif not first:
                        wait_w(nrb)
                    gds[k + 1] = pltpu.async_copy(
                        htab_h.at[ridp.at[slot, k + 1]], rows2.at[nrb],
                        gsems[nrb])
                gds[k].wait()

                def scale(r, _2, k=k, rb=rb, slot=slot):
                    fj = k * br + r
                    sidx = jnp.full((16,), slot * 4, I32) + (
                        lax.shift_right_logical(jnp.full((16,), fj, I32), 7))
                    eidx = jnp.full((16,), fj & 127, I32)
                    a16 = plsc.load_gather(alp, [sidx, eidx])
                    for g in range(cw):
                        for kc in range(8):
                            rows2[rb, r, g, pl.ds(kc * 16, 16)] = (
                                rows2[rb, r, g, pl.ds(kc * 16, 16)] * a16)
                    return _2
                lax.fori_loop(0, br, scale, 0)
                wds[k] = pltpu.async_copy(
                    rows2.at[rb], agg_acc.at[dllp.at[slot, k]], wsems[rb],
                    add=True)
            return wds

        def head_iter(hh, _h):
            head = c * hps + hh if heads == 8 else 0

            def zero_rows(i, _):
                for g in range(cw):
                    for kk in range(8):
                        rows2[0, i, g, pl.ds(kk * 16, 16)] = (
                            jnp.zeros((16,), F32))
                return _
            lax.fori_loop(0, br, zero_rows, 0)
            for k in range(nz):
                pltpu.sync_copy(rows2.at[0],
                                agg_acc.at[pl.ds(sid * rpt + k * br, br)])
            plsc.subcore_barrier()

            issue_stage(0, 0)
            issue_stage_ex(0, 0, head)
            wait_stage(0)
            issue_stage(1, 1)
            issue_stage_ex(1, 1, head)
            mkal(0, head)
            run_page(0, head, first=True)

            def pair(jj, _):
                for b, off in ((1, 1), (0, 2)):
                    pp = jj * 2 + off
                    wait_stage(b)
                    issue_stage(pp + 1, 1 - b)
                    issue_stage_ex(pp + 1, 1 - b, head)
                    mkal(b, head)
                    run_page(b, head, first=False)
                return _
            lax.fori_loop(0, (npg - 3) // 2, pair, 0)

            wait_stage(1)
            issue_stage(npg - 1, 0)
            issue_stage_ex(npg - 1, 0, head)
            mkal(1, head)
            run_page(1, head, first=False)
            wait_stage(0)
            mkal(0, head)
            wds = run_page(0, head, first=False)
            wds[nb - 2].wait()
            wds[nb - 1].wait()

            plsc.subcore_barrier()
            if heads == 8:
                pltpu.sync_copy(agg_acc.at[pl.ds(sid * rpt, rpt), 0],
                                agg_h.at[head, pl.ds(sid * rpt, rpt)])
            else:
                pltpu.sync_copy(agg_acc.at[pl.ds(sid * rpt, rpt)],
                                agg_h.at[pl.ds(nbase + sid * rpt, rpt)])
            plsc.subcore_barrier()
            return _h
        lax.fori_loop(0, hps, head_iter, 0)

    out_shape = (jax.ShapeDtypeStruct((heads, NP, ch), F32) if heads == 8
                 else jax.ShapeDtypeStruct((NP, cw, 128), F32))
    return pl.kernel(
        body,
        out_type=[out_shape],
        mesh=_mesh(),
        compiler_params=pltpu.CompilerParams(needs_layout_passes=False),
        scratch_types=[
            pltpu.VMEM((2 * pg,), I32),
            pltpu.VMEM((2 * pg,), I32),
            pltpu.VMEM((2 * pg,), F32),
            pltpu.VMEM((8, 128), F32),
            pltpu.VMEM((2, nb, br), I32),
            pltpu.VMEM((2, nb, br), I32),
            pltpu.VMEM((2, br, cw, 128), F32),
            pltpu.SemaphoreType.DMA,
            pltpu.SemaphoreType.DMA,
            pltpu.SemaphoreType.DMA,
            pltpu.SemaphoreType.DMA,
            pltpu.SemaphoreType.DMA,
            pltpu.VMEM_SHARED((accr, cw, 128), F32),
        ],
    )


def _make_rt_kernel(srows, combine):
    rcols = 1 if combine else srows
    BN = 256

    def body(s_ref, o_ref):
        s = s_ref[...]
        if combine:
            rr = 1.0 / (s[0:1] + s[1:2] + 1e-16)
        else:
            rr = 1.0 / (s + 1e-16)
        o_ref[...] = rr.T

    return pl.pallas_call(
        body,
        grid=(NP // BN,),
        in_specs=[pl.BlockSpec((srows, BN), lambda i: (0, i))],
        out_specs=pl.BlockSpec((BN, rcols), lambda i: (i, 0)),
        out_shape=jax.ShapeDtypeStruct((NP, rcols), F32),
    )


def _make_tc_first(dout, heads_out):
    BN = 256

    def body(x_ref, w_ref, as_ref, ad_ref, h_ref, at_s, at_d):
        h = jnp.dot(x_ref[...], w_ref[...], preferred_element_type=F32)
        h_ref[...] = h
        dn = (((0,), (1,)), ((), ()))
        at_s[...] = lax.dot_general(as_ref[...], h, dn,
                                    preferred_element_type=F32)
        at_d[...] = lax.dot_general(ad_ref[...], h, dn,
                                    preferred_element_type=F32)

    return pl.pallas_call(
        body,
        grid=(NP // BN,),
        in_specs=[pl.BlockSpec((BN, D_IN), lambda i: (i, 0)),
                  pl.BlockSpec((D_IN, dout), lambda i: (0, 0)),
                  pl.BlockSpec((dout, heads_out), lambda i: (0, 0)),
                  pl.BlockSpec((dout, heads_out), lambda i: (0, 0))],
        out_specs=[pl.BlockSpec((BN, dout), lambda i: (i, 0)),
                   pl.BlockSpec((heads_out, BN), lambda i: (0, i)),
                   pl.BlockSpec((heads_out, BN), lambda i: (0, i))],
        out_shape=[jax.ShapeDtypeStruct((NP, dout), F32),
                   jax.ShapeDtypeStruct((heads_out, NP), F32),
                   jax.ShapeDtypeStruct((heads_out, NP), F32)],
    )


def _make_tc_mid(dout, heads_out):
    BN = 256

    def body(a_ref, rt_ref, ge_ref, be_ref, w_ref, as_ref, ad_ref,
             h_ref, at_s, at_d):
        a = a_ref[...]
        rn = rt_ref[...]
        ge = ge_ref[...]
        be = be_ref[...]
        w = w_ref[...]
        h = None
        for k in range(HEADS):
            zk = a[k] * rn[:, k:k + 1]
            zk = jnp.maximum(zk * ge[k] + be[k], 0.0)
            hk = jnp.dot(zk, w[k * 128:(k + 1) * 128],
                         preferred_element_type=F32)
            h = hk if h is None else h + hk
        h_ref[...] = h
        dn = (((0,), (1,)), ((), ()))
        at_s[...] = lax.dot_general(as_ref[...], h, dn,
                                    preferred_element_type=F32)
        at_d[...] = lax.dot_general(ad_ref[...], h, dn,
                                    preferred_element_type=F32)

    return pl.pallas_call(
        body,
        grid=(NP // BN,),
        in_specs=[pl.BlockSpec((HEADS, BN, 128), lambda i: (0, i, 0)),
                  pl.BlockSpec((BN, HEADS), lambda i: (i, 0)),
                  pl.BlockSpec((HEADS, 1, 128), lambda i: (0, 0, 0)),
                  pl.BlockSpec((HEADS, 1, 128), lambda i: (0, 0, 0)),
                  pl.BlockSpec((HC, dout), lambda i: (0, 0)),
                  pl.BlockSpec((dout, heads_out), lambda i: (0, 0)),
                  pl.BlockSpec((dout, heads_out), lambda i: (0, 0))],
        out_specs=[pl.BlockSpec((BN, dout), lambda i: (i, 0)),
                   pl.BlockSpec((heads_out, BN), lambda i: (0, i)),
                   pl.BlockSpec((heads_out, BN), lambda i: (0, i))],
        out_shape=[jax.ShapeDtypeStruct((NP, dout), F32),
                   jax.ShapeDtypeStruct((heads_out, NP), F32),
                   jax.ShapeDtypeStruct((heads_out, NP), F32)],
    )


def _logsoftmax_kernel():
    BN = 256

    def body(y_ref, rt_ref, cb_ref, o_ref):
        y = y_ref[...] * rt_ref[...] + cb_ref[...]
        m = jnp.max(y, axis=-1, keepdims=True)
        z = y - m
        ssum = jnp.sum(jnp.exp(z), axis=-1, keepdims=True)
        o_ref[...] = z - jnp.log(ssum)

    return pl.pallas_call(
        body,
        grid=(NP // BN,),
        in_specs=[pl.BlockSpec((BN, OUT), lambda i: (i, 0)),
                  pl.BlockSpec((BN, 1), lambda i: (i, 0)),
                  pl.BlockSpec((1, OUT), lambda i: (0, 0))],
        out_specs=pl.BlockSpec((BN, OUT), lambda i: (i, 0)),
        out_shape=jax.ShapeDtypeStruct((NP, OUT), F32),
    )


def _block_diag_att(a):
    h, c = a.shape
    return (a[:, :, None] * jnp.eye(h, dtype=a.dtype)[:, None, :]).reshape(
        h * c, h)


def kernel(x, adj_t, W0, a_src0, a_dst0, b0, g0, be0,
           W1, a_src1, a_dst1, b1, g1, be1,
           W2, a_src2, a_dst2, b2, bias_last):
    adj = adj_t.astype(I32)
    loops = jnp.arange(N, dtype=I32)
    padv = jnp.full((EP - E - N,), N, dtype=I32)
    src = jnp.concatenate([adj[0], loops, padv])
    dst = jnp.concatenate([adj[1], loops, padv])
    xp = jnp.zeros((NP, D_IN), F32).at[:N].set(x)
    idm = jnp.arange(80, dtype=I32)

    bnscale = 1.0 / jnp.sqrt(jnp.float32(1.0 + 1e-5))
    ge0 = (g0 * bnscale).reshape(HEADS, 1, HID)
    bf0 = (b0 * g0 * bnscale + be0).reshape(HEADS, 1, HID)
    ge1 = (g1 * bnscale).reshape(HEADS, 1, HID)
    bf1 = (b1 * g1 * bnscale + be1).reshape(HEADS, 1, HID)
    cb = (b2 + bias_last).reshape(1, OUT)

    A_s0, A_d0 = _block_diag_att(a_src0), _block_diag_att(a_dst0)
    A_s1, A_d1 = _block_diag_att(a_src1), _block_diag_att(a_dst1)
    A_s2, A_d2 = a_src2.reshape(OUT, 1), a_dst2.reshape(OUT, 1)

    tc0 = _make_tc_first(HC, HEADS)
    tc1 = _make_tc_mid(HC, HEADS)
    tc2 = _make_tc_mid(OUT, 1)
    al8 = _make_alpha_kernel(8)
    al1 = _make_alpha_kernel(1)
    ag8 = _make_agg_kernel(8, HID)
    ag1 = _make_agg_kernel(1, OUT)
    rt8 = _make_rt_kernel(8, combine=False)
    rt1 = _make_rt_kernel(2, combine=True)
    lsm = _logsoftmax_kernel()

    h0, ast0, adt0 = tc0(xp, W0, A_s0, A_d0)
    ext0, st0 = al8(src, dst, ast0.reshape(8, 80, 128),
                    adt0.reshape(8, 80, 128), idm)
    (agg0,) = ag8(src, dst, ext0, h0.reshape(NP * HEADS, 1, HID))
    h1, ast1, adt1 = tc1(agg0, rt8(st0.reshape(8, NP)), ge0, bf0,
                         W1, A_s1, A_d1)
    ext1, st1 = al8(src, dst, ast1.reshape(8, 80, 128),
                    adt1.reshape(8, 80, 128), idm)
    (agg1,) = ag8(src, dst, ext1, h1.reshape(NP * HEADS, 1, HID))
    h2, ast2, adt2 = tc2(agg1, rt8(st1.reshape(8, NP)), ge1, bf1,
                         W2, A_s2, A_d2)
    ext2, st2 = al1(src, dst, ast2.reshape(1, 80, 128),
                    adt2.reshape(1, 80, 128), idm)
    (agg2,) = ag1(src, dst, ext2, h2.reshape(NP, 2, 128))
    out = lsm(agg2.reshape(NP, OUT), rt1(st2.reshape(2, NP)), cb)
    return out[:N]


# speedup vs baseline: 17.1055x; 1.0831x over previous
import functools

import jax
import jax.numpy as jnp
from jax import lax
from jax.experimental import pallas as pl
from jax.experimental.pallas import tpu as pltpu
from jax.experimental.pallas import tpu_sc as plsc

N = 10000
E = 160000
D_IN = 256
HID = 128
HEADS = 8
OUT = 256
HC = HEADS * HID

NP = 10240
EP = 172032
EPT = EP // 16
VE = EPT // 16
SB = EPT // 128
EPH = EP // 2
EPT2 = EPH // 16
VE2 = EPT2 // 16

F32 = jnp.float32
I32 = jnp.int32


def _mesh():
    return plsc.VectorSubcoreMesh(core_axis_name="c", subcore_axis_name="s",
                                  num_cores=2, num_subcores=16)


def _make_alpha_kernel(heads):
    hps = max(heads // 2, 1)
    ept = EPT if heads == 8 else EPT2
    ve = VE if heads == 8 else VE2

    def body(src_h, dst_h, ast_h, adt_h, idm_h, ext_h, st_h,
             srcb, dstb, asr, adr, sloc, exb, zsl, idxv, ssh):
        c = lax.axis_index("c")
        sid = lax.axis_index("s")
        base = sid * ept if heads == 8 else c * EPH + sid * ept
        pltpu.sync_copy(src_h.at[pl.ds(base, ept)], srcb)
        pltpu.sync_copy(dst_h.at[pl.ds(base, ept)], dstb)
        pltpu.sync_copy(idm_h, idxv)

        def zero_z(i, _):
            for kk in range(8):
                zsl[i, pl.ds(kk * 16, 16)] = jnp.zeros((16,), F32)
            return _
        lax.fori_loop(0, 8, zero_z, 0)

        def head_iter(hh, _h):
            head = c * hps + hh if heads == 8 else 0
            srow = head if heads == 8 else c
            pltpu.sync_copy(ast_h.at[head], asr)
            pltpu.sync_copy(adt_h.at[head], adr)

            def zero_s(i, _):
                for kk in range(8):
                    sloc[i, pl.ds(kk * 16, 16)] = jnp.zeros((16,), F32)
                return _
            lax.fori_loop(0, 80, zero_s, 0)

            def edge(i, _):
                s16 = srcb[pl.ds(i * 16, 16)]
                d16 = dstb[pl.ds(i * 16, 16)]
                si = [lax.shift_right_logical(s16, 7),
                      lax.bitwise_and(s16, 127)]
                di = [lax.shift_right_logical(d16, 7),
                      lax.bitwise_and(d16, 127)]
                av = plsc.load_gather(asr, si)
                dv = plsc.load_gather(adr, di)
                e = av + dv
                e = jnp.where(e < 0, e * F32(0.2), e)
                ex = jnp.exp(e)
                exb[pl.ds(i * 16, 16)] = ex
                plsc.addupdate_scatter(sloc, di, ex)
                return _
            lax.fori_loop(0, ve, edge, 0)

            pltpu.sync_copy(exb, ext_h.at[pl.ds(head * EP + base, ept)])

            @pl.when(sid < 10)
            def _():
                pltpu.sync_copy(zsl, ssh.at[pl.ds(sid * 8, 8)])
            plsc.subcore_barrier()
            pltpu.sync_copy(sloc, ssh.at[idxv], add=True)
            plsc.subcore_barrier()

            @pl.when(sid < 10)
            def _():
                pltpu.sync_copy(ssh.at[pl.ds(sid * 8, 8)],
                                st_h.at[srow, pl.ds(sid * 8, 8)])
            plsc.subcore_barrier()

    srows = heads if heads == 8 else 2
    return pl.kernel(
        body,
        out_type=[jax.ShapeDtypeStruct((heads * EP,), F32),
                  jax.ShapeDtypeStruct((srows, 80, 128), F32)],
        mesh=_mesh(),
        compiler_params=pltpu.CompilerParams(needs_layout_passes=False),
        scratch_types=[
            pltpu.VMEM((ept,), I32),
            pltpu.VMEM((ept,), I32),
            pltpu.VMEM((80, 128), F32),
            pltpu.VMEM((80, 128), F32),
            pltpu.VMEM((80, 128), F32),
            pltpu.VMEM((ept,), F32),
            pltpu.VMEM((8, 128), F32),
            pltpu.VMEM((80,), I32),
            pltpu.VMEM_SHARED((80, 128), F32),
        ],
    )


def _make_agg_kernel(heads, ch):
    hps = max(heads // 2, 1)
    accr = NP if heads == 8 else NP // 2
    rpt = accr // 16
    cw = ch // 128
    br = 128 // cw
    nz = rpt // br
    pg = 512
    nb = pg // br
    npg = EPT // pg
    npair = (npg - 1) // 2

    def body(src_h, dst_h, ext_h, htab_h, agg_h,
             srcp, dstp, exp_, alp, ridp, dllp, rows2,
             stsem, gsem0, gsem1, wsem0, wsem1, agg_acc):
        c = lax.axis_index("c")
        sid = lax.axis_index("s")
        ebase = sid * EPT
        nbase = c * accr if heads == 1 else 0
        gsems = (gsem0, gsem1)
        wsems = (wsem0, wsem1)

        def issue_stage(pp, slot):
            eoff = ebase + pp * pg
            so = slot * pg
            pltpu.async_copy(src_h.at[pl.ds(eoff, pg)],
                             srcp.at[pl.ds(so, pg)], stsem)
            pltpu.async_copy(dst_h.at[pl.ds(eoff, pg)],
                             dstp.at[pl.ds(so, pg)], stsem)

        def issue_stage_ex(pp, slot, head):
            eoff = ebase + pp * pg
            so = slot * pg
            pltpu.async_copy(ext_h.at[pl.ds(head * EP + eoff, pg)],
                             exp_.at[pl.ds(so, pg)], stsem)

        def wait_stage(slot):
            so = slot * pg
            pltpu.make_async_copy(src_h.at[pl.ds(0, pg)],
                                  srcp.at[pl.ds(so, pg)], stsem).wait()
            pltpu.make_async_copy(dst_h.at[pl.ds(0, pg)],
                                  dstp.at[pl.ds(so, pg)], stsem).wait()
            pltpu.make_async_copy(ext_h.at[pl.ds(0, pg)],
                                  exp_.at[pl.ds(so, pg)], stsem).wait()

        def wait_w(rb):
            pltpu.make_async_copy(rows2.at[rb], agg_acc.at[ridp.at[0, 0]],
                                  wsems[rb]).wait()

        def mkal(slot, head):
            so = slot * pg

            def step(i, _):
                s16 = srcp[pl.ds(so + i * 16, 16)]
                d16 = dstp[pl.ds(so + i * 16, 16)]
                a16 = exp_[pl.ds(so + i * 16, 16)]
                rid = s16 * heads + head
                if heads == 1:
                    inb = (d16 >= nbase) & (d16 < nbase + accr)
                    a16 = jnp.where(inb, a16, F32(0.0))
                    dloc = jnp.clip(d16 - nbase, 0, accr - 1)
                else:
                    dloc = d16
                alp[slot * 4 + i // 8, pl.ds((i % 8) * 16, 16)] = a16
                kb = i // (br // 16)
                ko = (i % (br // 16)) * 16
                ridp[slot, kb, pl.ds(ko, 16)] = rid
                dllp[slot, kb, pl.ds(ko, 16)] = dloc
                return _
            lax.fori_loop(0, pg // 16, step, 0)

        def run_page(slot, head, first):
            wds = [None] * nb
            if not first:
                wait_w(0)
            gds = [None] * nb
            gds[0] = pltpu.async_copy(
                htab_h.at[ridp.at[slot, 0]], rows2.at[0], gsems[0])
            for k in range(nb):
                rb = k % 2
                if k + 1 < nb:
                    nrb = 1 - rb
                    if k + 1 >= 2:
                        wds[k - 1].wait()
                    elif not first:
                        wait_w(nrb)
                    gds[k + 1] = pltpu.async_copy(
                        htab_h.at[ridp.at[slot, k + 1]], rows2.at[nrb],
                        gsems[nrb])
                gds[k].wait()

                def aget(fj, slot=slot):
                    row = slot * 4 + lax.shift_right_logical(fj, 7)
                    col = lax.bitwise_and(fj, 127)
                    return plsc.load_gather(
                        alp, [jnp.full((16,), row, I32),
                              jnp.full((16,), col, I32)])

                def scale(r, ac, k=k, rb=rb):
                    an = aget(k * br + jnp.minimum(r + 1, br - 1))
                    for g in range(cw):
                        for kc in range(8):
                            rows2[rb, r, g, pl.ds(kc * 16, 16)] = (
                                rows2[rb, r, g, pl.ds(kc * 16, 16)] * ac)
                    return an
                lax.fori_loop(0, br, scale, aget(jnp.int32(k * br)))
                wds[k] = pltpu.async_copy(
                    rows2.at[rb], agg_acc.at[dllp.at[slot, k]], wsems[rb],
                    add=True)
            return wds

        def head_iter(hh, _h):
            head = c * hps + hh if heads == 8 else 0

            def zero_rows(i, _):
                for g in range(cw):
                    for kk in range(8):
                        rows2[0, i, g, pl.ds(kk * 16, 16)] = (
                            jnp.zeros((16,), F32))
                return _
            lax.fori_loop(0, br, zero_rows, 0)
            for k in range(nz):
                pltpu.sync_copy(rows2.at[0],
                                agg_acc.at[pl.ds(sid * rpt + k * br, br)])
            plsc.subcore_barrier()

            issue_stage(0, 0)
            issue_stage_ex(0, 0, head)
            wait_stage(0)
            issue_stage(1, 1)
            issue_stage_ex(1, 1, head)
            mkal(0, head)
            run_page(0, head, first=True)

            def pair(jj, _):
                for b, off in ((1, 1), (0, 2)):
                    pp = jj * 2 + off
                    wait_stage(b)
                    issue_stage(pp + 1, 1 - b)
                    issue_stage_ex(pp + 1, 1 - b, head)
                    mkal(b, head)
                    run_page(b, head, first=False)
                return _
            lax.fori_loop(0, (npg - 3) // 2, pair, 0)

            wait_stage(1)
            issue_stage(npg - 1, 0)
            issue_stage_ex(npg - 1, 0, head)
            mkal(1, head)
            run_page(1, head, first=False)
            wait_stage(0)
            mkal(0, head)
            wds = run_page(0, head, first=False)
            wds[nb - 2].wait()
            wds[nb - 1].wait()

            plsc.subcore_barrier()
            if heads == 8:
                pltpu.sync_copy(agg_acc.at[pl.ds(sid * rpt, rpt), 0],
                                agg_h.at[head, pl.ds(sid * rpt, rpt)])
            else:
                pltpu.sync_copy(agg_acc.at[pl.ds(sid * rpt, rpt)],
                                agg_h.at[pl.ds(nbase + sid * rpt, rpt)])
            plsc.subcore_barrier()
            return _h
        lax.fori_loop(0, hps, head_iter, 0)

    out_shape = (jax.ShapeDtypeStruct((heads, NP, ch), F32) if heads == 8
                 else jax.ShapeDtypeStruct((NP, cw, 128), F32))
    return pl.kernel(
        body,
        out_type=[out_shape],
        mesh=_mesh(),
        compiler_params=pltpu.CompilerParams(needs_layout_passes=False),
        scratch_types=[
            pltpu.VMEM((2 * pg,), I32),
            pltpu.VMEM((2 * pg,), I32),
            pltpu.VMEM((2 * pg,), F32),
            pltpu.VMEM((8, 128), F32),
            pltpu.VMEM((2, nb, br), I32),
            pltpu.VMEM((2, nb, br), I32),
            pltpu.VMEM((2, br, cw, 128), F32),
            pltpu.SemaphoreType.DMA,
            pltpu.SemaphoreType.DMA,
            pltpu.SemaphoreType.DMA,
            pltpu.SemaphoreType.DMA,
            pltpu.SemaphoreType.DMA,
            pltpu.VMEM_SHARED((accr, cw, 128), F32),
        ],
    )


def _make_rt_kernel(srows, combine):
    rcols = 1 if combine else srows
    BN = 256

    def body(s_ref, o_ref):
        s = s_ref[...]
        if combine:
            rr = 1.0 / (s[0:1] + s[1:2] + 1e-16)
        else:
            rr = 1.0 / (s + 1e-16)
        o_ref[...] = rr.T

    return pl.pallas_call(
        body,
        grid=(NP // BN,),
        in_specs=[pl.BlockSpec((srows, BN), lambda i: (0, i))],
        out_specs=pl.BlockSpec((BN, rcols), lambda i: (i, 0)),
        out_shape=jax.ShapeDtypeStruct((NP, rcols), F32),
    )


def _make_tc_first(dout, heads_out):
    BN = 256

    def body(x_ref, w_ref, as_ref, ad_ref, h_ref, at_s, at_d):
        h = jnp.dot(x_ref[...], w_ref[...], preferred_element_type=F32)
        h_ref[...] = h
        dn = (((0,), (1,)), ((), ()))
        at_s[...] = lax.dot_general(as_ref[...], h, dn,
                                    preferred_element_type=F32)
        at_d[...] = lax.dot_general(ad_ref[...], h, dn,
                                    preferred_element_type=F32)

    return pl.pallas_call(
        body,
        grid=(NP // BN,),
        in_specs=[pl.BlockSpec((BN, D_IN), lambda i: (i, 0)),
                  pl.BlockSpec((D_IN, dout), lambda i: (0, 0)),
                  pl.BlockSpec((dout, heads_out), lambda i: (0, 0)),
                  pl.BlockSpec((dout, heads_out), lambda i: (0, 0))],
        out_specs=[pl.BlockSpec((BN, dout), lambda i: (i, 0)),
                   pl.BlockSpec((heads_out, BN), lambda i: (0, i)),
                   pl.BlockSpec((heads_out, BN), lambda i: (0, i))],
        out_shape=[jax.ShapeDtypeStruct((NP, dout), F32),
                   jax.ShapeDtypeStruct((heads_out, NP), F32),
                   jax.ShapeDtypeStruct((heads_out, NP), F32)],
    )


def _make_tc_mid(dout, heads_out):
    BN = 256

    def body(a_ref, rt_ref, ge_ref, be_ref, w_ref, as_ref, ad_ref,
             h_ref, at_s, at_d):
        a = a_ref[...]
        rn = rt_ref[...]
        ge = ge_ref[...]
        be = be_ref[...]
        w = w_ref[...]
        h = None
        for k in range(HEADS):
            zk = a[k] * rn[:, k:k + 1]
            zk = jnp.maximum(zk * ge[k] + be[k], 0.0)
            hk = jnp.dot(zk, w[k * 128:(k + 1) * 128],
                         preferred_element_type=F32)
            h = hk if h is None else h + hk
        h_ref[...] = h
        dn = (((0,), (1,)), ((), ()))
        at_s[...] = lax.dot_general(as_ref[...], h, dn,
                                    preferred_element_type=F32)
        at_d[...] = lax.dot_general(ad_ref[...], h, dn,
                                    preferred_element_type=F32)

    return pl.pallas_call(
        body,
        grid=(NP // BN,),
        in_specs=[pl.BlockSpec((HEADS, BN, 128), lambda i: (0, i, 0)),
                  pl.BlockSpec((BN, HEADS), lambda i: (i, 0)),
                  pl.BlockSpec((HEADS, 1, 128), lambda i: (0, 0, 0)),
                  pl.BlockSpec((HEADS, 1, 128), lambda i: (0, 0, 0)),
                  pl.BlockSpec((HC, dout), lambda i: (0, 0)),
                  pl.BlockSpec((dout, heads_out), lambda i: (0, 0)),
                  pl.BlockSpec((dout, heads_out), lambda i: (0, 0))],
        out_specs=[pl.BlockSpec((BN, dout), lambda i: (i, 0)),
                   pl.BlockSpec((heads_out, BN), lambda i: (0, i)),
                   pl.BlockSpec((heads_out, BN), lambda i: (0, i))],
        out_shape=[jax.ShapeDtypeStruct((NP, dout), F32),
                   jax.ShapeDtypeStruct((heads_out, NP), F32),
                   jax.ShapeDtypeStruct((heads_out, NP), F32)],
    )


def _logsoftmax_kernel():
    BN = 256

    def body(y_ref, rt_ref, cb_ref, o_ref):
        y = y_ref[...] * rt_ref[...] + cb_ref[...]
        m = jnp.max(y, axis=-1, keepdims=True)
        z = y - m
        ssum = jnp.sum(jnp.exp(z), axis=-1, keepdims=True)
        o_ref[...] = z - jnp.log(ssum)

    return pl.pallas_call(
        body,
        grid=(NP // BN,),
        in_specs=[pl.BlockSpec((BN, OUT), lambda i: (i, 0)),
                  pl.BlockSpec((BN, 1), lambda i: (i, 0)),
                  pl.BlockSpec((1, OUT), lambda i: (0, 0))],
        out_specs=pl.BlockSpec((BN, OUT), lambda i: (i, 0)),
        out_shape=jax.ShapeDtypeStruct((NP, OUT), F32),
    )


def _block_diag_att(a):
    h, c = a.shape
    return (a[:, :, None] * jnp.eye(h, dtype=a.dtype)[:, None, :]).reshape(
        h * c, h)


def kernel(x, adj_t, W0, a_src0, a_dst0, b0, g0, be0,
           W1, a_src1, a_dst1, b1, g1, be1,
           W2, a_src2, a_dst2, b2, bias_last):
    adj = adj_t.astype(I32)
    loops = jnp.arange(N, dtype=I32)
    padv = jnp.full((EP - E - N,), N, dtype=I32)
    src = jnp.concatenate([adj[0], loops, padv])
    dst = jnp.concatenate([adj[1], loops, padv])
    xp = jnp.zeros((NP, D_IN), F32).at[:N].set(x)
    idm = jnp.arange(80, dtype=I32)

    bnscale = 1.0 / jnp.sqrt(jnp.float32(1.0 + 1e-5))
    ge0 = (g0 * bnscale).reshape(HEADS, 1, HID)
    bf0 = (b0 * g0 * bnscale + be0).reshape(HEADS, 1, HID)
    ge1 = (g1 * bnscale).reshape(HEADS, 1, HID)
    bf1 = (b1 * g1 * bnscale + be1).reshape(HEADS, 1, HID)
    cb = (b2 + bias_last).reshape(1, OUT)

    A_s0, A_d0 = _block_diag_att(a_src0), _block_diag_att(a_dst0)
    A_s1, A_d1 = _block_diag_att(a_src1), _block_diag_att(a_dst1)
    A_s2, A_d2 = a_src2.reshape(OUT, 1), a_dst2.reshape(OUT, 1)

    tc0 = _make_tc_first(HC, HEADS)
    tc1 = _make_tc_mid(HC, HEADS)
    tc2 = _make_tc_mid(OUT, 1)
    al8 = _make_alpha_kernel(8)
    al1 = _make_alpha_kernel(1)
    ag8 = _make_agg_kernel(8, HID)
    ag1 = _make_agg_kernel(1, OUT)
    rt8 = _make_rt_kernel(8, combine=False)
    rt1 = _make_rt_kernel(2, combine=True)
    lsm = _logsoftmax_kernel()

    h0, ast0, adt0 = tc0(xp, W0, A_s0, A_d0)
    ext0, st0 = al8(src, dst, ast0.reshape(8, 80, 128),
                    adt0.reshape(8, 80, 128), idm)
    (agg0,) = ag8(src, dst, ext0, h0.reshape(NP * HEADS, 1, HID))
    h1, ast1, adt1 = tc1(agg0, rt8(st0.reshape(8, NP)), ge0, bf0,
                         W1, A_s1, A_d1)
    ext1, st1 = al8(src, dst, ast1.reshape(8, 80, 128),
                    adt1.reshape(8, 80, 128), idm)
    (agg1,) = ag8(src, dst, ext1, h1.reshape(NP * HEADS, 1, HID))
    h2, ast2, adt2 = tc2(agg1, rt8(st1.reshape(8, NP)), ge1, bf1,
                         W2, A_s2, A_d2)
    ext2, st2 = al1(src, dst, ast2.reshape(1, 80, 128),
                    adt2.reshape(1, 80, 128), idm)
    (agg2,) = ag1(src, dst, ext2, h2.reshape(NP, 2, 128))
    out = lsm(agg2.reshape(NP, OUT), rt1(st2.reshape(2, NP)), cb)
    return out[:N]
